# baseline (speedup 1.0000x reference)
import functools

import jax
import jax.numpy as jnp
from jax import lax
from jax.experimental import pallas as pl
from jax.experimental.pallas import tpu as pltpu
from jax.experimental.pallas import tpu_sc as plsc


_NC, _NS = 2, 16
_NW = _NC * _NS


def _gather_x(emb, idx2, B, E):
    bpw = B // _NW
    mesh = plsc.VectorSubcoreMesh(core_axis_name="c", subcore_axis_name="s")

    half = bpw // 2

    @functools.partial(
        pl.kernel,
        mesh=mesh,
        out_type=jax.ShapeDtypeStruct((B, E), jnp.float32),
        scratch_types=[
            pltpu.VMEM((2, half), jnp.int32),
            pltpu.VMEM((half, E), jnp.float32),
            pltpu.VMEM((half, E), jnp.float32),
            pltpu.SemaphoreType.DMA,
            pltpu.SemaphoreType.DMA,
            pltpu.SemaphoreType.DMA,
        ],
    )
    def k(emb_hbm, idx_hbm, out_hbm, idx_v, buf0, buf1, g0, g1, so):
        wid = lax.axis_index("s") * _NC + lax.axis_index("c")
        base = wid * bpw
        pltpu.sync_copy(idx_hbm.at[wid], idx_v)
        pltpu.async_copy(emb_hbm.at[idx_v.at[0]], buf0, g0)
        pltpu.async_copy(emb_hbm.at[idx_v.at[1]], buf1, g1)
        pltpu.make_async_copy(emb_hbm.at[idx_v.at[0]], buf0, g0).wait()
        pltpu.async_copy(buf0, out_hbm.at[pl.ds(base, half)], so)
        pltpu.make_async_copy(emb_hbm.at[idx_v.at[1]], buf1, g1).wait()
        pltpu.sync_copy(buf1, out_hbm.at[pl.ds(base + half, half)])
        pltpu.make_async_copy(buf0, out_hbm.at[pl.ds(base, half)], so).wait()

    return k(emb, idx2)


def _cast_body(w_ref, o_ref):
    o_ref[...] = w_ref[...].astype(jnp.bfloat16)


def _cast_bf16(W):
    O, E = W.shape
    BO = 1024
    return pl.pallas_call(
        _cast_body,
        grid=(O // BO,),
        in_specs=[pl.BlockSpec((BO, E), lambda j: (j, 0))],
        out_specs=pl.BlockSpec((BO, E), lambda j: (j, 0)),
        out_shape=jax.ShapeDtypeStruct((O, E), jnp.bfloat16),
    )(W)


def _mm_body(w_ref, x_ref, b_ref, o_ref, xbf_ref):
    @pl.when(pl.program_id(0) == 0)
    def _():
        xbf_ref[...] = x_ref[...].astype(jnp.bfloat16)

    y = jax.lax.dot_general(
        w_ref[...], xbf_ref[...],
        dimension_numbers=(((1,), (1,)), ((), ())),
        preferred_element_type=jnp.float32,
    )
    y = y + b_ref[...]
    o_ref[...] = 0.5 * y * (1.0 + jnp.tanh(0.5 * y))


def _mm_silu_t(Wbf, x, b2):
    O, E = Wbf.shape
    B = x.shape[0]
    BO = 512
    return pl.pallas_call(
        _mm_body,
        grid=(O // BO,),
        in_specs=[
            pl.BlockSpec((BO, E), lambda j: (j, 0)),
            pl.BlockSpec((B, E), lambda j: (0, 0)),
            pl.BlockSpec((BO, 1), lambda j: (j, 0)),
        ],
        out_specs=pl.BlockSpec((BO, B), lambda j: (j, 0)),
        out_shape=jax.ShapeDtypeStruct((O, B), jnp.float32),
        scratch_shapes=[pltpu.VMEM((B, E), jnp.bfloat16)],
    )(Wbf, x, b2)


def kernel(label, emb_table, W, b):
    B = label.shape[0]
    E = emb_table.shape[1]
    O = W.shape[0]
    idx2 = jnp.reshape(label.astype(jnp.int32), (_NW, 2, B // _NW // 2))
    Wbf = _cast_bf16(W)
    Wbf, idx2 = jax.lax.optimization_barrier((Wbf, idx2))
    x = _gather_x(emb_table, idx2, B, E)
    outT = _mm_silu_t(Wbf, x, jnp.reshape(b, (O, 1)))
    return jnp.reshape(outT.T, (B, 1, 4, 32, 32))


# speedup vs baseline: 1.0226x; 1.0226x over previous
import functools

import jax
import jax.numpy as jnp
from jax import lax
from jax.experimental import pallas as pl
from jax.experimental.pallas import tpu as pltpu
from jax.experimental.pallas import tpu_sc as plsc


_NC, _NS = 2, 16
_NW = _NC * _NS


def _gather_x(emb, idx2, B, E):
    bpw = B // _NW
    mesh = plsc.VectorSubcoreMesh(core_axis_name="c", subcore_axis_name="s")

    @functools.partial(
        pl.kernel,
        mesh=mesh,
        out_type=jax.ShapeDtypeStruct((B, E), jnp.float32),
        scratch_types=[
            pltpu.VMEM((bpw,), jnp.int32),
            pltpu.VMEM((bpw, E), jnp.float32),
            pltpu.SemaphoreType.DMA,
        ],
    )
    def k(emb_hbm, idx_hbm, out_hbm, idx_v, rows_v, sem):
        wid = lax.axis_index("s") * _NC + lax.axis_index("c")
        pltpu.sync_copy(idx_hbm.at[wid], idx_v)
        pltpu.async_copy(emb_hbm.at[idx_v], rows_v, sem).wait()
        pltpu.sync_copy(rows_v, out_hbm.at[pl.ds(wid * bpw, bpw)])

    return k(emb, idx2)


def _cast_body(w_ref, o_ref):
    o_ref[...] = w_ref[...].astype(jnp.bfloat16)


def _cast_bf16(W):
    O, E = W.shape
    BO = 1024
    return pl.pallas_call(
        _cast_body,
        grid=(O // BO,),
        in_specs=[pl.BlockSpec((BO, E), lambda j: (j, 0))],
        out_specs=pl.BlockSpec((BO, E), lambda j: (j, 0)),
        out_shape=jax.ShapeDtypeStruct((O, E), jnp.bfloat16),
    )(W)


def _mm_body(w_ref, x_ref, b_ref, o_ref, xbf_ref):
    @pl.when(pl.program_id(0) == 0)
    def _():
        xbf_ref[...] = x_ref[...].astype(jnp.bfloat16)

    y = jax.lax.dot_general(
        w_ref[...], xbf_ref[...],
        dimension_numbers=(((1,), (1,)), ((), ())),
        preferred_element_type=jnp.float32,
    )
    y = y + b_ref[...]
    o_ref[...] = 0.5 * y * (1.0 + jnp.tanh(0.5 * y))


def _mm_silu_t(Wbf, x, b2):
    O, E = Wbf.shape
    B = x.shape[0]
    BO = 512
    return pl.pallas_call(
        _mm_body,
        grid=(O // BO,),
        in_specs=[
            pl.BlockSpec((BO, E), lambda j: (j, 0)),
            pl.BlockSpec((B, E), lambda j: (0, 0)),
            pl.BlockSpec((BO, 1), lambda j: (j, 0)),
        ],
        out_specs=pl.BlockSpec((BO, B), lambda j: (j, 0)),
        out_shape=jax.ShapeDtypeStruct((O, B), jnp.float32),
        scratch_shapes=[pltpu.VMEM((B, E), jnp.bfloat16)],
    )(Wbf, x, b2)


def kernel(label, emb_table, W, b):
    B = label.shape[0]
    E = emb_table.shape[1]
    O = W.shape[0]
    idx2 = jnp.reshape(label.astype(jnp.int32), (_NW, B // _NW))
    Wbf = _cast_bf16(W)
    Wbf, idx2 = jax.lax.optimization_barrier((Wbf, idx2))
    x = _gather_x(emb_table, idx2, B, E)
    outT = _mm_silu_t(Wbf, x, jnp.reshape(b, (O, 1)))
    return jnp.reshape(outT.T, (B, 1, 4, 32, 32))
